# baseline (speedup 1.0000x reference)
import jax
import jax.numpy as jnp
from jax import lax
from jax.experimental import pallas as pl
from jax.experimental.pallas import tpu as pltpu


def _make_body(v: int, n: int):
    def body(idx_ref, tab_ref, x_ref, o_ref, pet_ref):
        @pl.when(pl.program_id(0) == 0)
        def _():
            idx = idx_ref[0, :]
            iota = lax.broadcasted_iota(jnp.int32, (v, n), 0)
            oh = jnp.where(iota == idx[None, :], 1.0, 0.0)
            pet_ref[...] = jnp.dot(tab_ref[...], oh,
                                   preferred_element_type=jnp.float32,
                                   precision=lax.Precision.HIGHEST)

        o_ref[...] = x_ref[...] + pet_ref[...][None, :, :]

    return body


def kernel(x, input_channels, pos_embed):
    B, N, P, E = x.shape
    V = pos_embed.shape[0]
    idx2 = input_channels.astype(jnp.int32).reshape(1, N)
    tab_t = pos_embed.T
    xt = jnp.transpose(x, (0, 2, 3, 1)).reshape(B * P, E, N)

    CH = 50
    out_t = pl.pallas_call(
        _make_body(V, N),
        grid=(B * P // CH,),
        in_specs=[
            pl.BlockSpec((1, N), lambda c: (0, 0)),
            pl.BlockSpec((E, V), lambda c: (0, 0)),
            pl.BlockSpec((CH, E, N), lambda c: (c, 0, 0)),
        ],
        out_specs=pl.BlockSpec((CH, E, N), lambda c: (c, 0, 0)),
        out_shape=jax.ShapeDtypeStruct((B * P, E, N), jnp.float32),
        scratch_shapes=[pltpu.VMEM((E, N), jnp.float32)],
    )(idx2, tab_t, xt)
    return jnp.transpose(out_t.reshape(B, P, E, N), (0, 3, 1, 2))


# speedup vs baseline: 1.0164x; 1.0164x over previous
import jax
import jax.numpy as jnp
from jax import lax
from jax.experimental import pallas as pl
from jax.experimental.pallas import tpu as pltpu


def _make_body(v: int, n: int):
    def body(idx_ref, tab_ref, x_ref, o_ref, pet_ref):
        @pl.when(pl.program_id(0) == 0)
        def _():
            idx = idx_ref[0, :]
            iota = lax.broadcasted_iota(jnp.int32, (v, n), 0)
            oh = jnp.where(iota == idx[None, :], 1.0, 0.0)
            pet_ref[...] = jnp.dot(tab_ref[...], oh,
                                   preferred_element_type=jnp.float32,
                                   precision=lax.Precision.HIGHEST)

        o_ref[...] = x_ref[...] + pet_ref[...][None, :, :]

    return body


def kernel(x, input_channels, pos_embed):
    B, N, P, E = x.shape
    V = pos_embed.shape[0]
    idx2 = input_channels.astype(jnp.int32).reshape(1, N)
    tab_t = pos_embed.T
    xt = jnp.transpose(x, (0, 2, 3, 1)).reshape(B * P, E, N)

    CH = 100
    out_t = pl.pallas_call(
        _make_body(V, N),
        grid=(B * P // CH,),
        in_specs=[
            pl.BlockSpec((1, N), lambda c: (0, 0)),
            pl.BlockSpec((E, V), lambda c: (0, 0)),
            pl.BlockSpec((CH, E, N), lambda c: (c, 0, 0)),
        ],
        out_specs=pl.BlockSpec((CH, E, N), lambda c: (c, 0, 0)),
        out_shape=jax.ShapeDtypeStruct((B * P, E, N), jnp.float32),
        scratch_shapes=[pltpu.VMEM((E, N), jnp.float32)],
    )(idx2, tab_t, xt)
    return jnp.transpose(out_t.reshape(B, P, E, N), (0, 3, 1, 2))
